# baseline (speedup 1.0000x reference)
import functools

import jax
import jax.numpy as jnp
from jax import lax
from jax.experimental import pallas as pl
from jax.experimental.pallas import tpu as pltpu
from jax.experimental.pallas import tpu_sc as plsc

_NC = 2
_NS = 16
_NW = _NC * _NS
_LANES = 16


def _make_dispatch(rows, h, rpw, nch, ch):
    mesh = plsc.VectorSubcoreMesh(core_axis_name="c", subcore_axis_name="s")

    @functools.partial(
        pl.kernel,
        out_type=jax.ShapeDtypeStruct((rows, h), jnp.float32),
        mesh=mesh,
        scratch_types=(
            [pltpu.VMEM((ch,), jnp.int32) for _ in range(nch)]
            + [pltpu.VMEM((ch, h), jnp.float32) for _ in range(nch)]
            + [pltpu.SemaphoreType.DMA for _ in range(nch)]
            + [pltpu.SemaphoreType.DMA]
        ),
    )
    def dispatch(x_hbm, idx_hbm, xs_hbm, *sc):
        idxs = sc[:nch]
        bufs = sc[nch:2 * nch]
        gsems = sc[2 * nch:3 * nch]
        ssem = sc[3 * nch]
        wid = lax.axis_index("s") * _NC + lax.axis_index("c")
        base = wid * rpw
        for c in range(nch):
            pltpu.sync_copy(idx_hbm.at[wid, c], idxs[c])
        cps = [pltpu.async_copy(x_hbm.at[idxs[c]], bufs[c], gsems[c])
               for c in range(nch)]
        sts = []
        for c in range(nch):
            cps[c].wait()
            sts.append(pltpu.async_copy(
                bufs[c], xs_hbm.at[pl.ds(base + c * ch, ch)], ssem))
        for st in sts:
            st.wait()

    return dispatch


def _make_combine(t_all, h, tpw, ncht, ct):
    mesh = plsc.VectorSubcoreMesh(core_axis_name="c", subcore_axis_name="s")

    @functools.partial(
        pl.kernel,
        out_type=jax.ShapeDtypeStruct((t_all, h), jnp.float32),
        mesh=mesh,
        scratch_types=(
            [pltpu.VMEM((ct,), jnp.int32) for _ in range(2 * ncht)]
            + [pltpu.VMEM((ct, h), jnp.float32) for _ in range(2 * ncht)]
            + [pltpu.SemaphoreType.DMA for _ in range(2 * ncht)]
            + [pltpu.SemaphoreType.DMA]
        ),
    )
    def combine(ys_hbm, idx0_hbm, idx1_hbm, out_hbm, *sc):
        idxs = sc[:2 * ncht]
        bufs = sc[2 * ncht:4 * ncht]
        gsems = sc[4 * ncht:6 * ncht]
        ssem = sc[6 * ncht]
        wid = lax.axis_index("s") * _NC + lax.axis_index("c")
        nsl = h // _LANES
        for c in range(ncht):
            pltpu.sync_copy(idx0_hbm.at[wid, c], idxs[2 * c])
            pltpu.sync_copy(idx1_hbm.at[wid, c], idxs[2 * c + 1])
        cps = [pltpu.async_copy(ys_hbm.at[idxs[j]], bufs[j], gsems[j])
               for j in range(2 * ncht)]
        sts = []
        for c in range(ncht):
            r0_v, r1_v = bufs[2 * c], bufs[2 * c + 1]
            cps[2 * c].wait()
            cps[2 * c + 1].wait()

            def row_add(i, _, r0_v=r0_v, r1_v=r1_v):
                for s in range(nsl):
                    sl = pl.ds(s * _LANES, _LANES)
                    r0_v[i, sl] = r0_v[i, sl] + r1_v[i, sl]
                return 0

            lax.fori_loop(0, ct, row_add, 0)
            sts.append(pltpu.async_copy(
                r0_v, out_hbm.at[pl.ds(wid * tpw + c * ct, ct)], ssem))
        for st in sts:
            st.wait()

    return combine


def _gmm_body(be_ref, bo_ref, bv_ref, xs_ref, gw_ref, uw_ref, dw_ref,
              rw_ref, ys_ref):
    b = pl.program_id(0)

    @pl.when(bv_ref[b] == 1)
    def _():
        xb = xs_ref[...]
        g = lax.dot_general(xb, gw_ref[0], (((1,), (1,)), ((), ())),
                            preferred_element_type=jnp.float32)
        u = lax.dot_general(xb, uw_ref[0], (((1,), (1,)), ((), ())),
                            preferred_element_type=jnp.float32)
        hact = (g * jax.nn.sigmoid(g)) * u
        hact = hact * rw_ref[...]
        ys_ref[...] = lax.dot_general(hact, dw_ref[0],
                                      (((1,), (0,)), ((), ())),
                                      preferred_element_type=jnp.float32)


def _grouped_ffn(be, bo, bv, xs, gw, uw, dw, rws, blk, nb, rows):
    e, i_dim, h = gw.shape
    grid_spec = pltpu.PrefetchScalarGridSpec(
        num_scalar_prefetch=3,
        grid=(nb,),
        in_specs=[
            pl.BlockSpec((blk, h), lambda b, be, bo, bv: (bo[b], 0)),
            pl.BlockSpec((1, i_dim, h), lambda b, be, bo, bv: (be[b], 0, 0)),
            pl.BlockSpec((1, i_dim, h), lambda b, be, bo, bv: (be[b], 0, 0)),
            pl.BlockSpec((1, i_dim, h), lambda b, be, bo, bv: (be[b], 0, 0)),
            pl.BlockSpec((blk, 1), lambda b, be, bo, bv: (bo[b], 0)),
        ],
        out_specs=pl.BlockSpec((blk, h), lambda b, be, bo, bv: (bo[b], 0)),
    )
    return pl.pallas_call(
        _gmm_body,
        grid_spec=grid_spec,
        out_shape=jax.ShapeDtypeStruct((rows, h), jnp.float32),
    )(be, bo, bv, xs, gw, uw, dw, rws)


def kernel(gathered_experts_out_buf, x, output_sample, input_sample,
           selected_experts, routing_weights, num_select_experts,
           selected_experts_middle, routing_weights_middle, block_size,
           num_experts, num_experts_per_tok, use_grouped_topk,
           num_expert_group, topk_group, gate_weights, up_weights,
           down_weights):
    t_all, h = x.shape
    e, i_dim, _ = gate_weights.shape
    k = selected_experts.shape[1]
    r = t_all * k

    blk = 128
    nb = r // blk + e
    rows = nb * blk

    sel = selected_experts.reshape(-1).astype(jnp.int32)
    rwf = routing_weights.reshape(-1).astype(jnp.float32)
    eids = jnp.arange(e, dtype=jnp.int32)
    onehot = (sel[:, None] == eids[None, :]).astype(jnp.int32)
    csum = jnp.cumsum(onehot, axis=0)
    counts = csum[-1]
    rank = jnp.take_along_axis(csum - 1, sel[:, None], axis=1)[:, 0]
    padded = ((counts + blk - 1) // blk) * blk
    pad_off = jnp.concatenate(
        [jnp.zeros((1,), jnp.int32), jnp.cumsum(padded)[:-1].astype(jnp.int32)])
    pos = pad_off[sel] + rank
    nvalid = (jnp.sum(padded) // blk).astype(jnp.int32)
    tok = jnp.arange(r, dtype=jnp.int32) // k
    src_tok = jnp.zeros((rows,), jnp.int32).at[pos].set(tok)
    rws = jnp.zeros((rows,), jnp.float32).at[pos].set(rwf)

    bid = jnp.arange(nb, dtype=jnp.int32)
    start_blk = (pad_off // blk).astype(jnp.int32)
    bo = jnp.minimum(bid, nvalid - 1)
    be_raw = jnp.sum(
        (bid[:, None] >= start_blk[None, :]).astype(jnp.int32), axis=1) - 1
    be = be_raw[bo]
    bv = (bid < nvalid).astype(jnp.int32)

    rpw = rows // _NW
    nch = 4
    ch = rpw // nch
    dispatch = _make_dispatch(rows, h, rpw, nch, ch)
    xs = dispatch(x, src_tok.reshape(_NW, nch, ch))

    ys = _grouped_ffn(be, bo, bv, xs, gate_weights, up_weights,
                      down_weights, rws[:, None], blk, nb, rows)

    tpw = t_all // _NW
    ncht = 2
    ct = tpw // ncht
    pos2 = pos.reshape(t_all, k)
    idx0 = pos2[:, 0].reshape(_NW, ncht, ct)
    idx1 = pos2[:, 1].reshape(_NW, ncht, ct)
    combine = _make_combine(t_all, h, tpw, ncht, ct)
    return combine(ys, idx0, idx1)


# speedup vs baseline: 1.4458x; 1.4458x over previous
import functools

import jax
import jax.numpy as jnp
from jax import lax
from jax.experimental import pallas as pl
from jax.experimental.pallas import tpu as pltpu
from jax.experimental.pallas import tpu_sc as plsc

_NC = 2
_NS = 16
_NW = _NC * _NS
_LANES = 16


def _make_dispatch(rows, h, tpw):
    mesh = plsc.VectorSubcoreMesh(core_axis_name="c", subcore_axis_name="s")

    @functools.partial(
        pl.kernel,
        out_type=jax.ShapeDtypeStruct((rows, h), jnp.float32),
        mesh=mesh,
        scratch_types=[
            pltpu.VMEM((tpw,), jnp.int32),
            pltpu.VMEM((tpw,), jnp.int32),
            pltpu.VMEM((tpw, h), jnp.float32),
            pltpu.SemaphoreType.DMA,
            pltpu.SemaphoreType.DMA,
            pltpu.SemaphoreType.DMA,
        ],
    )
    def dispatch(x_hbm, idx0_hbm, idx1_hbm, xs_hbm,
                 idx0_v, idx1_v, buf, semg, sem0, sem1):
        wid = lax.axis_index("s") * _NC + lax.axis_index("c")
        base = wid * tpw
        pltpu.sync_copy(idx0_hbm.at[wid], idx0_v)
        pltpu.sync_copy(idx1_hbm.at[wid], idx1_v)
        pltpu.async_copy(x_hbm.at[pl.ds(base, tpw)], buf, semg).wait()
        cp0 = pltpu.async_copy(buf, xs_hbm.at[idx0_v], sem0)
        cp1 = pltpu.async_copy(buf, xs_hbm.at[idx1_v], sem1)
        cp0.wait()
        cp1.wait()

    return dispatch


def _make_combine(t_all, h, tpw, ncht, ct):
    mesh = plsc.VectorSubcoreMesh(core_axis_name="c", subcore_axis_name="s")

    @functools.partial(
        pl.kernel,
        out_type=jax.ShapeDtypeStruct((t_all, h), jnp.float32),
        mesh=mesh,
        scratch_types=(
            [pltpu.VMEM((ct,), jnp.int32) for _ in range(2 * ncht)]
            + [pltpu.VMEM((ct, h), jnp.float32) for _ in range(2 * ncht)]
            + [pltpu.SemaphoreType.DMA for _ in range(2 * ncht)]
            + [pltpu.SemaphoreType.DMA]
        ),
    )
    def combine(ys_hbm, idx0_hbm, idx1_hbm, out_hbm, *sc):
        idxs = sc[:2 * ncht]
        bufs = sc[2 * ncht:4 * ncht]
        gsems = sc[4 * ncht:6 * ncht]
        ssem = sc[6 * ncht]
        wid = lax.axis_index("s") * _NC + lax.axis_index("c")
        nsl = h // _LANES
        for c in range(ncht):
            pltpu.sync_copy(idx0_hbm.at[wid, c], idxs[2 * c])
            pltpu.sync_copy(idx1_hbm.at[wid, c], idxs[2 * c + 1])
        cps = [pltpu.async_copy(ys_hbm.at[idxs[j]], bufs[j], gsems[j])
               for j in range(2 * ncht)]
        sts = []
        for c in range(ncht):
            r0_v, r1_v = bufs[2 * c], bufs[2 * c + 1]
            cps[2 * c].wait()
            cps[2 * c + 1].wait()

            def row_add(i, _, r0_v=r0_v, r1_v=r1_v):
                for s in range(nsl):
                    sl = pl.ds(s * _LANES, _LANES)
                    r0_v[i, sl] = r0_v[i, sl] + r1_v[i, sl]
                return 0

            lax.fori_loop(0, ct, row_add, 0)
            sts.append(pltpu.async_copy(
                r0_v, out_hbm.at[pl.ds(wid * tpw + c * ct, ct)], ssem))
        for st in sts:
            st.wait()

    return combine


def _gmm_body(be_ref, bo_ref, bv_ref, xs_ref, gw_ref, uw_ref, dw_ref,
              rw_ref, ys_ref):
    b = pl.program_id(0)

    @pl.when(bv_ref[b] == 1)
    def _():
        xb = xs_ref[...]
        g = lax.dot_general(xb, gw_ref[0], (((1,), (1,)), ((), ())),
                            preferred_element_type=jnp.float32)
        u = lax.dot_general(xb, uw_ref[0], (((1,), (1,)), ((), ())),
                            preferred_element_type=jnp.float32)
        hact = (g * jax.nn.sigmoid(g)) * u
        hact = hact * rw_ref[...]
        ys_ref[...] = lax.dot_general(hact, dw_ref[0],
                                      (((1,), (0,)), ((), ())),
                                      preferred_element_type=jnp.float32)


def _grouped_ffn(be, bo, bv, xs, gw, uw, dw, rws, blk, nb, rows):
    e, i_dim, h = gw.shape
    grid_spec = pltpu.PrefetchScalarGridSpec(
        num_scalar_prefetch=3,
        grid=(nb,),
        in_specs=[
            pl.BlockSpec((blk, h), lambda b, be, bo, bv: (bo[b], 0)),
            pl.BlockSpec((1, i_dim, h), lambda b, be, bo, bv: (be[b], 0, 0)),
            pl.BlockSpec((1, i_dim, h), lambda b, be, bo, bv: (be[b], 0, 0)),
            pl.BlockSpec((1, i_dim, h), lambda b, be, bo, bv: (be[b], 0, 0)),
            pl.BlockSpec((blk, 1), lambda b, be, bo, bv: (bo[b], 0)),
        ],
        out_specs=pl.BlockSpec((blk, h), lambda b, be, bo, bv: (bo[b], 0)),
    )
    return pl.pallas_call(
        _gmm_body,
        grid_spec=grid_spec,
        out_shape=jax.ShapeDtypeStruct((rows, h), jnp.float32),
    )(be, bo, bv, xs, gw, uw, dw, rws)


def kernel(gathered_experts_out_buf, x, output_sample, input_sample,
           selected_experts, routing_weights, num_select_experts,
           selected_experts_middle, routing_weights_middle, block_size,
           num_experts, num_experts_per_tok, use_grouped_topk,
           num_expert_group, topk_group, gate_weights, up_weights,
           down_weights):
    t_all, h = x.shape
    e, i_dim, _ = gate_weights.shape
    k = selected_experts.shape[1]
    r = t_all * k

    blk = 128
    nb = r // blk + e
    rows = nb * blk

    sel = selected_experts.reshape(-1).astype(jnp.int32)
    rwf = routing_weights.reshape(-1).astype(jnp.float32)
    eids = jnp.arange(e, dtype=jnp.int32)
    onehot = (sel[:, None] == eids[None, :]).astype(jnp.int32)
    csum = jnp.cumsum(onehot, axis=0)
    counts = csum[-1]
    rank = jnp.take_along_axis(csum - 1, sel[:, None], axis=1)[:, 0]
    padded = ((counts + blk - 1) // blk) * blk
    pad_off = jnp.concatenate(
        [jnp.zeros((1,), jnp.int32), jnp.cumsum(padded)[:-1].astype(jnp.int32)])
    pos = pad_off[sel] + rank
    nvalid = (jnp.sum(padded) // blk).astype(jnp.int32)
    rws = jnp.zeros((rows,), jnp.float32).at[pos].set(rwf)

    bid = jnp.arange(nb, dtype=jnp.int32)
    start_blk = (pad_off // blk).astype(jnp.int32)
    bo = jnp.minimum(bid, nvalid - 1)
    be_raw = jnp.sum(
        (bid[:, None] >= start_blk[None, :]).astype(jnp.int32), axis=1) - 1
    be = be_raw[bo]
    bv = (bid < nvalid).astype(jnp.int32)

    tpw = t_all // _NW
    pos2 = pos.reshape(t_all, k)
    dispatch = _make_dispatch(rows, h, tpw)
    xs = dispatch(x, pos2[:, 0].reshape(_NW, tpw),
                  pos2[:, 1].reshape(_NW, tpw))

    ys = _grouped_ffn(be, bo, bv, xs, gate_weights, up_weights,
                      down_weights, rws[:, None], blk, nb, rows)

    ncht = 2
    ct = tpw // ncht
    idx0 = pos2[:, 0].reshape(_NW, ncht, ct)
    idx1 = pos2[:, 1].reshape(_NW, ncht, ct)
    combine = _make_combine(t_all, h, tpw, ncht, ct)
    return combine(ys, idx0, idx1)
